# baseline (speedup 1.0000x reference)
import jax
import jax.numpy as jnp
from jax.experimental import pallas as pl

_TOKENS = 2048
_D_IN = 768
_BASE = 128
_BT = 128


def _pkm_kernel(x_ref, w_ref, b_ref, o_ref):
    xs = jax.lax.dot_general(
        x_ref[:], w_ref[:],
        (((1,), (1,)), ((), ())),
        preferred_element_type=jnp.float32,
    ) + b_ref[:]
    x1 = xs[:, :_BASE]
    x2 = xs[:, _BASE:]
    o_ref[:] = x1[:, :, None] + x2[:, None, :]


def kernel(x, W, b):
    b2 = b.reshape(1, 2 * _BASE)
    out3 = pl.pallas_call(
        _pkm_kernel,
        grid=(_TOKENS // _BT,),
        in_specs=[
            pl.BlockSpec((_BT, _D_IN), lambda t: (t, 0)),
            pl.BlockSpec((2 * _BASE, _D_IN), lambda t: (0, 0)),
            pl.BlockSpec((1, 2 * _BASE), lambda t: (0, 0)),
        ],
        out_specs=pl.BlockSpec((_BT, _BASE, _BASE), lambda t: (t, 0, 0)),
        out_shape=jax.ShapeDtypeStruct((_TOKENS, _BASE, _BASE), jnp.float32),
    )(x, W, b2)
    return out3.reshape(_TOKENS, _BASE * _BASE)


# speedup vs baseline: 2.9624x; 2.9624x over previous
import jax
import jax.numpy as jnp
from jax.experimental import pallas as pl

_TOKENS = 2048
_D_IN = 768
_BASE = 128
_BT = 128


def _pkm_kernel(x_ref, w_ref, b_ref, o_ref):
    xs = jax.lax.dot_general(
        x_ref[:], w_ref[:],
        (((1,), (1,)), ((), ())),
        preferred_element_type=jnp.float32,
    ) + b_ref[:]
    x1 = xs[:, :_BASE]
    x2 = xs[:, _BASE:]
    for i in range(_BASE):
        o_ref[:, i * _BASE:(i + 1) * _BASE] = x1[:, i:i + 1] + x2


def kernel(x, W, b):
    b2 = b.reshape(1, 2 * _BASE)
    return pl.pallas_call(
        _pkm_kernel,
        grid=(_TOKENS // _BT,),
        in_specs=[
            pl.BlockSpec((_BT, _D_IN), lambda t: (t, 0)),
            pl.BlockSpec((2 * _BASE, _D_IN), lambda t: (0, 0)),
            pl.BlockSpec((1, 2 * _BASE), lambda t: (0, 0)),
        ],
        out_specs=pl.BlockSpec((_BT, _BASE * _BASE), lambda t: (t, 0)),
        out_shape=jax.ShapeDtypeStruct((_TOKENS, _BASE * _BASE), jnp.float32),
    )(x, W, b2)


# speedup vs baseline: 3.1161x; 1.0519x over previous
import jax
import jax.numpy as jnp
from jax.experimental import pallas as pl

_TOKENS = 2048
_D_IN = 768
_BASE = 128
_BT = 256


def _pkm_kernel(x_ref, w_ref, b_ref, o_ref):
    xs = jax.lax.dot_general(
        x_ref[:], w_ref[:],
        (((1,), (1,)), ((), ())),
        preferred_element_type=jnp.float32,
    ) + b_ref[:]
    x1 = xs[:, :_BASE]
    x2 = xs[:, _BASE:]
    for i in range(_BASE):
        o_ref[:, i * _BASE:(i + 1) * _BASE] = x1[:, i:i + 1] + x2


def kernel(x, W, b):
    b2 = b.reshape(1, 2 * _BASE)
    return pl.pallas_call(
        _pkm_kernel,
        grid=(_TOKENS // _BT,),
        in_specs=[
            pl.BlockSpec((_BT, _D_IN), lambda t: (t, 0)),
            pl.BlockSpec((2 * _BASE, _D_IN), lambda t: (0, 0)),
            pl.BlockSpec((1, 2 * _BASE), lambda t: (0, 0)),
        ],
        out_specs=pl.BlockSpec((_BT, _BASE * _BASE), lambda t: (t, 0)),
        out_shape=jax.ShapeDtypeStruct((_TOKENS, _BASE * _BASE), jnp.float32),
    )(x, W, b2)
